# baseline (speedup 1.0000x reference)
import jax
import jax.numpy as jnp
from jax.experimental import pallas as pl
from jax.experimental.pallas import tpu as pltpu

_B, _S, _D = 4, 8192, 768
_K = 128
_N = _B * _S

_TN = 2048
_NB = _N // _TN
_CH = 4096


def _fused_kernel(mem_ref, proj_ref, out_ref, mn_ref, a_ref, s1_ref, u3_ref):
    i = pl.program_id(0)

    @pl.when(i == 0)
    def _():
        m = mem_ref[...]
        sq = jnp.sum(m * m, axis=1, keepdims=True)
        mn_ref[...] = m * jax.lax.rsqrt(jnp.maximum(sq, 1e-12))
        s1_ref[...] = jnp.zeros_like(s1_ref)

    @pl.when(i < _NB)
    def _():
        p = proj_ref[...]
        sq = jnp.sum(p * p, axis=1, keepdims=True)
        pn = p * jax.lax.rsqrt(jnp.maximum(sq, 1e-12))
        sim = jnp.dot(pn, mn_ref[...].T, preferred_element_type=jnp.float32)
        a = jnp.exp(sim / 0.05)
        a_ref[pl.ds(i * _TN, _TN), :] = a
        s1_ref[...] += jnp.sum(a, axis=0, keepdims=True)

    @pl.when(i == _NB)
    def _():
        def sink(u):
            def body(c, acc):
                a = a_ref[pl.ds(c * _CH, _CH), :]
                t = jax.lax.dot_general(a, u, (((1,), (1,)), ((), ())),
                                        preferred_element_type=jnp.float32)
                v = 1.0 / (_N * t)
                return acc + jax.lax.dot_general(v, a, (((0,), (0,)), ((), ())),
                                                 preferred_element_type=jnp.float32)
            s = jax.lax.fori_loop(0, _N // _CH, body,
                                  jnp.zeros((1, _K), jnp.float32))
            return 1.0 / (_K * s)

        u1 = 1.0 / (_K * s1_ref[...])
        u2 = sink(u1)
        u3_ref[...] = sink(u2)

    @pl.when(i >= _NB)
    def _():
        j = i - _NB
        a = a_ref[pl.ds(j * _TN, _TN), :]
        scores = a * u3_ref[...]
        idx = jnp.argmax(scores, axis=1)
        iota = jax.lax.broadcasted_iota(jnp.int32, scores.shape, 1)
        onehot = (iota == idx[:, None]).astype(jnp.float32)
        assign = jnp.dot(onehot, mem_ref[...], preferred_element_type=jnp.float32)
        out_ref[...] = (proj_ref[...] + assign) * 0.5


@jax.jit
def kernel(projections, memory):
    bsz, seq, d = projections.shape
    proj = projections.reshape(-1, d)
    wmem = memory[:_K, :]

    out = pl.pallas_call(
        _fused_kernel,
        grid=(2 * _NB,),
        in_specs=[
            pl.BlockSpec((_K, _D), lambda i: (0, 0)),
            pl.BlockSpec((_TN, _D), lambda i: (jnp.where(i < _NB, i, i - _NB), 0)),
        ],
        out_specs=pl.BlockSpec((_TN, _D),
                               lambda i: (jnp.where(i < _NB, 0, i - _NB), 0)),
        out_shape=jax.ShapeDtypeStruct((_N, _D), jnp.float32),
        scratch_shapes=[
            pltpu.VMEM((_K, _D), jnp.float32),
            pltpu.VMEM((_N, _K), jnp.float32),
            pltpu.VMEM((1, _K), jnp.float32),
            pltpu.VMEM((1, _K), jnp.float32),
        ],
    )(wmem, proj)

    return out.reshape(bsz, seq, d)


# speedup vs baseline: 1.0827x; 1.0827x over previous
import jax
import jax.numpy as jnp
from jax.experimental import pallas as pl
from jax.experimental.pallas import tpu as pltpu

_B, _S, _D = 4, 8192, 768
_K = 128
_N = _B * _S

_TN = 2048
_NB = _N // _TN
_CH = 2048
_NBF = 4


def _fused_kernel(mem_ref, proj_ref, out_ref, mn_ref, a_ref, pb_ref, s1_ref,
                  u3_ref):
    i = pl.program_id(0)

    @pl.when(i == 0)
    def _():
        m = mem_ref[...]
        sq = jnp.sum(m * m, axis=1, keepdims=True)
        mn_ref[...] = m * jax.lax.rsqrt(jnp.maximum(sq, 1e-12))
        s1_ref[...] = jnp.zeros_like(s1_ref)

    @pl.when(i < _NB)
    def _():
        p = proj_ref[...]
        sq = jnp.sum(p * p, axis=1, keepdims=True)
        pn = p * jax.lax.rsqrt(jnp.maximum(sq, 1e-12))
        sim = jnp.dot(pn, mn_ref[...].T, preferred_element_type=jnp.float32)
        a = jnp.exp(sim / 0.05)
        a_ref[pl.ds(i * _TN, _TN), :] = a
        s1_ref[...] += jnp.sum(a, axis=0, keepdims=True)

        @pl.when(i >= _NB - _NBF)
        def _():
            pb_ref[pl.ds((i - (_NB - _NBF)) * _TN, _TN), :] = p.astype(jnp.bfloat16)

    @pl.when(i == _NB)
    def _():
        def sink(u):
            def body(c, acc):
                a = a_ref[pl.ds(c * _CH, _CH), :]
                t = jnp.sum(a * u, axis=1, keepdims=True)
                v = 1.0 / (_N * t)
                return acc + jnp.sum(a * v, axis=0, keepdims=True)
            s = jax.lax.fori_loop(0, _N // _CH, body,
                                  jnp.zeros((1, _K), jnp.float32))
            return 1.0 / (_K * s)

        u1 = 1.0 / (_K * s1_ref[...])
        u2 = sink(u1)
        u3_ref[...] = sink(u2)

    @pl.when(i >= _NB)
    def _():
        j = i - _NB
        a = a_ref[pl.ds(j * _TN, _TN), :]
        scores = a * u3_ref[...]
        idx = jnp.argmax(scores, axis=1)
        iota = jax.lax.broadcasted_iota(jnp.int32, scores.shape, 1)
        onehot = (iota == idx[:, None]).astype(jnp.float32)
        assign = jnp.dot(onehot, mem_ref[...], preferred_element_type=jnp.float32)

        @pl.when(j < _NB - _NBF)
        def _():
            out_ref[...] = (proj_ref[...] + assign) * 0.5

        @pl.when(j >= _NB - _NBF)
        def _():
            p = pb_ref[pl.ds((j - (_NB - _NBF)) * _TN, _TN), :].astype(jnp.float32)
            out_ref[...] = (p + assign) * 0.5


@jax.jit
def kernel(projections, memory):
    bsz, seq, d = projections.shape
    proj = projections.reshape(-1, d)
    wmem = memory[:_K, :]

    out = pl.pallas_call(
        _fused_kernel,
        grid=(2 * _NB,),
        in_specs=[
            pl.BlockSpec((_K, _D), lambda i: (0, 0)),
            pl.BlockSpec(
                (_TN, _D),
                lambda i: (jnp.where(i < _NB, i,
                                     jnp.minimum(i - _NB, _NB - _NBF - 1)), 0)),
        ],
        out_specs=pl.BlockSpec((_TN, _D),
                               lambda i: (jnp.where(i < _NB, 0, i - _NB), 0)),
        out_shape=jax.ShapeDtypeStruct((_N, _D), jnp.float32),
        compiler_params=pltpu.CompilerParams(
            vmem_limit_bytes=100 * 1024 * 1024),
        scratch_shapes=[
            pltpu.VMEM((_K, _D), jnp.float32),
            pltpu.VMEM((_N, _K), jnp.float32),
            pltpu.VMEM((_NBF * _TN, _D), jnp.bfloat16),
            pltpu.VMEM((1, _K), jnp.float32),
            pltpu.VMEM((1, _K), jnp.float32),
        ],
    )(wmem, proj)

    return out.reshape(bsz, seq, d)


# speedup vs baseline: 1.1014x; 1.0173x over previous
import jax
import jax.numpy as jnp
from jax.experimental import pallas as pl
from jax.experimental.pallas import tpu as pltpu

_B, _S, _D = 4, 8192, 768
_K = 128
_N = _B * _S

_TN = 2048
_NB = _N // _TN
_CH = 2048
_NBF = 5


def _fused_kernel(mem_ref, proj_ref, out_ref, mn_ref, a_ref, pb_ref, s1_ref,
                  u3_ref):
    i = pl.program_id(0)

    @pl.when(i == 0)
    def _():
        m = mem_ref[...]
        sq = jnp.sum(m * m, axis=1, keepdims=True)
        mn_ref[...] = m * jax.lax.rsqrt(jnp.maximum(sq, 1e-12))
        s1_ref[...] = jnp.zeros_like(s1_ref)

    @pl.when(i < _NB)
    def _():
        p = proj_ref[...]
        sq = jnp.sum(p * p, axis=1, keepdims=True)
        pn = p * jax.lax.rsqrt(jnp.maximum(sq, 1e-12))
        sim = jnp.dot(pn, mn_ref[...].T, preferred_element_type=jnp.float32)
        a = jnp.exp(sim / 0.05)
        a_ref[pl.ds(i * _TN, _TN), :] = a
        s1_ref[...] += jnp.sum(a, axis=0, keepdims=True)

        @pl.when(i >= _NB - _NBF)
        def _():
            pb_ref[pl.ds((i - (_NB - _NBF)) * _TN, _TN), :] = p.astype(jnp.bfloat16)

    @pl.when(i == _NB)
    def _():
        def sink(u):
            def body(c, acc):
                a = a_ref[pl.ds(c * _CH, _CH), :]
                t = jnp.sum(a * u, axis=1, keepdims=True)
                v = 1.0 / (_N * t)
                return acc + jnp.sum(a * v, axis=0, keepdims=True)
            s = jax.lax.fori_loop(0, _N // _CH, body,
                                  jnp.zeros((1, _K), jnp.float32))
            return 1.0 / (_K * s)

        u1 = 1.0 / (_K * s1_ref[...])
        u2 = sink(u1)
        u3_ref[...] = sink(u2)

    @pl.when(i >= _NB)
    def _():
        j = i - _NB
        a = a_ref[pl.ds(j * _TN, _TN), :]
        scores = a * u3_ref[...]
        idx = jnp.argmax(scores, axis=1)
        iota = jax.lax.broadcasted_iota(jnp.int32, scores.shape, 1)
        onehot = (iota == idx[:, None]).astype(jnp.float32)
        assign = jnp.dot(onehot, mem_ref[...], preferred_element_type=jnp.float32)

        @pl.when(j < _NB - _NBF)
        def _():
            out_ref[...] = (proj_ref[...] + assign) * 0.5

        @pl.when(j >= _NB - _NBF)
        def _():
            p = pb_ref[pl.ds((j - (_NB - _NBF)) * _TN, _TN), :].astype(jnp.float32)
            out_ref[...] = (p + assign) * 0.5


@jax.jit
def kernel(projections, memory):
    bsz, seq, d = projections.shape
    proj = projections.reshape(-1, d)
    wmem = memory[:_K, :]

    out = pl.pallas_call(
        _fused_kernel,
        grid=(2 * _NB,),
        in_specs=[
            pl.BlockSpec((_K, _D), lambda i: (0, 0)),
            pl.BlockSpec(
                (_TN, _D),
                lambda i: (jnp.where(i < _NB, i,
                                     jnp.minimum(i - _NB, _NB - _NBF - 1)), 0)),
        ],
        out_specs=pl.BlockSpec((_TN, _D),
                               lambda i: (jnp.where(i < _NB, 0, i - _NB), 0)),
        out_shape=jax.ShapeDtypeStruct((_N, _D), jnp.float32),
        compiler_params=pltpu.CompilerParams(
            vmem_limit_bytes=100 * 1024 * 1024),
        scratch_shapes=[
            pltpu.VMEM((_K, _D), jnp.float32),
            pltpu.VMEM((_N, _K), jnp.float32),
            pltpu.VMEM((_NBF * _TN, _D), jnp.bfloat16),
            pltpu.VMEM((1, _K), jnp.float32),
            pltpu.VMEM((1, _K), jnp.float32),
        ],
    )(wmem, proj)

    return out.reshape(bsz, seq, d)


# speedup vs baseline: 1.1162x; 1.0135x over previous
import jax
import jax.numpy as jnp
from jax.experimental import pallas as pl
from jax.experimental.pallas import tpu as pltpu

_B, _S, _D = 4, 8192, 768
_K = 128
_N = _B * _S

_TN = 2048
_NB = _N // _TN
_CH = 4096
_NBF = 5


def _fused_kernel(mem_ref, proj_ref, out_ref, mn_ref, a_ref, pb_ref, s1_ref,
                  u3_ref):
    i = pl.program_id(0)

    @pl.when(i == 0)
    def _():
        m = mem_ref[...]
        sq = jnp.sum(m * m, axis=1, keepdims=True)
        mn_ref[...] = m * jax.lax.rsqrt(jnp.maximum(sq, 1e-12))
        s1_ref[...] = jnp.zeros_like(s1_ref)

    @pl.when(i < _NB)
    def _():
        p = proj_ref[...]
        sq = jnp.sum(p * p, axis=1, keepdims=True)
        pn = p * jax.lax.rsqrt(jnp.maximum(sq, 1e-12))
        sim = jnp.dot(pn, mn_ref[...].T, preferred_element_type=jnp.float32)
        a = jnp.exp(sim / 0.05)
        a_ref[pl.ds(i * _TN, _TN), :] = a
        s1_ref[...] += jnp.sum(a, axis=0, keepdims=True)

        @pl.when(i >= _NB - _NBF)
        def _():
            pb_ref[pl.ds((i - (_NB - _NBF)) * _TN, _TN), :] = p.astype(jnp.bfloat16)

    @pl.when(i == _NB)
    def _():
        def sink(u):
            def body(c, acc):
                a = a_ref[pl.ds(c * _CH, _CH), :]
                t = jnp.sum(a * u, axis=1, keepdims=True)
                v = 1.0 / (_N * t)
                return acc + jnp.sum(a * v, axis=0, keepdims=True)
            s = jax.lax.fori_loop(0, _N // _CH, body,
                                  jnp.zeros((1, _K), jnp.float32))
            return 1.0 / (_K * s)

        u1 = 1.0 / (_K * s1_ref[...])
        u2 = sink(u1)
        u3_ref[...] = sink(u2)

    @pl.when(i >= _NB)
    def _():
        j = i - _NB
        a = a_ref[pl.ds(j * _TN, _TN), :]
        scores = a * u3_ref[...]
        idx = jnp.argmax(scores, axis=1)
        iota = jax.lax.broadcasted_iota(jnp.int32, scores.shape, 1)
        onehot = (iota == idx[:, None]).astype(jnp.float32)
        assign = jnp.dot(onehot, mem_ref[...], preferred_element_type=jnp.float32)

        @pl.when(j < _NB - _NBF)
        def _():
            out_ref[...] = (proj_ref[...] + assign) * 0.5

        @pl.when(j >= _NB - _NBF)
        def _():
            p = pb_ref[pl.ds((j - (_NB - _NBF)) * _TN, _TN), :].astype(jnp.float32)
            out_ref[...] = (p + assign) * 0.5


@jax.jit
def kernel(projections, memory):
    bsz, seq, d = projections.shape
    proj = projections.reshape(-1, d)
    wmem = memory[:_K, :]

    out = pl.pallas_call(
        _fused_kernel,
        grid=(2 * _NB,),
        in_specs=[
            pl.BlockSpec((_K, _D), lambda i: (0, 0)),
            pl.BlockSpec(
                (_TN, _D),
                lambda i: (jnp.where(i < _NB, i,
                                     jnp.minimum(i - _NB, _NB - _NBF - 1)), 0)),
        ],
        out_specs=pl.BlockSpec((_TN, _D),
                               lambda i: (jnp.where(i < _NB, 0, i - _NB), 0)),
        out_shape=jax.ShapeDtypeStruct((_N, _D), jnp.float32),
        compiler_params=pltpu.CompilerParams(
            vmem_limit_bytes=100 * 1024 * 1024),
        scratch_shapes=[
            pltpu.VMEM((_K, _D), jnp.float32),
            pltpu.VMEM((_N, _K), jnp.float32),
            pltpu.VMEM((_NBF * _TN, _D), jnp.bfloat16),
            pltpu.VMEM((1, _K), jnp.float32),
            pltpu.VMEM((1, _K), jnp.float32),
        ],
    )(wmem, proj)

    return out.reshape(bsz, seq, d)
